# baseline (speedup 1.0000x reference)
import functools

import jax
import jax.numpy as jnp
from jax.experimental import pallas as pl
from jax.experimental.pallas import tpu as pltpu
from jax.experimental.pallas import tpu_sc as plsc

_K = 32
_L = 200
_NEG_INF = float("-inf")


def _topk_body(tw_ref, gate_ref, w_ref, pos_out_ref):
    tw = tw_ref[...]
    gate = gate_ref[...]
    l, c = tw.shape

    pos = jax.lax.broadcasted_iota(jnp.int32, (l, c), 0)
    s = jnp.sum(gate, axis=0, keepdims=True)
    need = s < _K
    keep = (pos >= 1) & (pos <= _K)
    unmask = (gate != 0) | (keep & need)
    twm = jnp.where(unmask, tw, _NEG_INF)

    j = pl.program_id(0)
    lane1 = jax.lax.broadcasted_iota(jnp.int32, (1, c), 1)
    flatbase = (j * c + lane1) * _L

    big = jnp.int32(1 << 30)
    for k in range(_K):
        m = jnp.max(twm, axis=0, keepdims=True)
        eq = twm == m
        minp = jnp.min(jnp.where(eq, pos, big), axis=0, keepdims=True)
        sel = pos == minp
        twm = jnp.where(sel, _NEG_INF, twm)
        w_ref[pl.ds(k, 1), :] = m
        pos_out_ref[pl.ds(k, 1), :] = flatbase + minp

    vals = w_ref[...]
    e = jnp.exp(vals - vals[0:1, :])
    w_ref[...] = e / jnp.sum(e, axis=0, keepdims=True)


_NC = 2
_NS = 16
_NW = _NC * _NS


_IW = 128


def _sc_gather_body(tid_hbm, attn_hbm, idx_hbm, tid_out, attn_out,
                    idxv, otidv, oattnv, sem, irows_w, irows_t):
    wid = jax.lax.axis_index("s") * _NC + jax.lax.axis_index("c")
    irow0 = wid * irows_w
    n_tiles = irows_w // irows_t

    def tile_body(t, carry):
        rb = irow0 + t * irows_t
        pltpu.sync_copy(idx_hbm.at[pl.ds(rb, irows_t)], idxv)
        hs = [pltpu.async_copy(tid_hbm.at[idxv.at[j]], otidv.at[j], sem)
              for j in range(irows_t)]
        for h in hs:
            h.wait()
        hs = [pltpu.async_copy(attn_hbm.at[idxv.at[j]], oattnv.at[j], sem)
              for j in range(irows_t)]
        for h in hs:
            h.wait()
        pltpu.sync_copy(otidv, tid_out.at[pl.ds(rb, irows_t)])
        pltpu.sync_copy(oattnv, attn_out.at[pl.ds(rb, irows_t)])
        return carry

    jax.lax.fori_loop(0, n_tiles, tile_body, 0)


def _sc_gather(token_id_flat, attn_flat, idx2d, b):
    irows = (b * _K) // _IW
    irows_w = irows // _NW
    irows_t = min(irows_w, 16)
    mesh = plsc.VectorSubcoreMesh(core_axis_name="c", subcore_axis_name="s")
    body = functools.partial(_sc_gather_body, irows_w=irows_w, irows_t=irows_t)
    tid2d, attn2d = pl.kernel(
        body,
        mesh=mesh,
        out_type=[
            jax.ShapeDtypeStruct((irows, _IW), jnp.int32),
            jax.ShapeDtypeStruct((irows, _IW), jnp.int32),
        ],
        scratch_types=[
            pltpu.VMEM((irows_t, _IW), jnp.int32),
            pltpu.VMEM((irows_t, _IW), jnp.int32),
            pltpu.VMEM((irows_t, _IW), jnp.int32),
            pltpu.SemaphoreType.DMA,
        ],
    )(token_id_flat, attn_flat, idx2d)
    return tid2d, attn2d


def kernel(token_id, attn_mask, gate_mask, token_weight):
    b, l = token_weight.shape
    c = min(2048, b)
    grid = (b // c,)

    tw_t = token_weight.T
    gate_t = gate_mask.T

    in_spec = pl.BlockSpec((l, c), lambda j: (0, j))
    out_spec = pl.BlockSpec((_K, c), lambda j: (0, j))

    w_t, pos_t = pl.pallas_call(
        _topk_body,
        grid=grid,
        in_specs=[in_spec, in_spec],
        out_specs=[out_spec, out_spec],
        out_shape=[
            jax.ShapeDtypeStruct((_K, b), jnp.float32),
            jax.ShapeDtypeStruct((_K, b), jnp.int32),
        ],
    )(tw_t, gate_t)

    idx2d = pos_t.T.reshape(-1, _IW)
    tid2d, attn2d = _sc_gather(
        token_id.reshape(b * l), attn_mask.reshape(b * l), idx2d, b)

    return (tid2d.reshape(b, _K), attn2d.reshape(b, _K), w_t.T)


# speedup vs baseline: 2.2884x; 2.2884x over previous
import jax
import jax.numpy as jnp
from jax.experimental import pallas as pl

_K = 32
_L = 200
_NEG_INF = float("-inf")


def _topk_body(tw_ref, tid_ref, gate_ref, attn_ref, w_ref, tid_out_ref, attn_out_ref):
    tw = tw_ref[...]
    tid = tid_ref[...]
    gate = gate_ref[...]
    attn = attn_ref[...]
    l, c = tw.shape

    pos = jax.lax.broadcasted_iota(jnp.int32, (l, c), 0)
    packed = pos * 65536 + attn * 32768 + tid
    s = jnp.sum(gate, axis=0, keepdims=True)
    need = s < _K
    keep = (pos >= 1) & (pos <= _K)
    unmask = (gate != 0) | (keep & need)
    twm = jnp.where(unmask, tw, _NEG_INF)

    big = jnp.int32(1 << 30)
    for k in range(_K):
        m = jnp.max(twm, axis=0, keepdims=True)
        eq = twm == m
        minp = jnp.min(jnp.where(eq, packed, big), axis=0, keepdims=True)
        sel = packed == minp
        twm = jnp.where(sel, _NEG_INF, twm)
        w_ref[pl.ds(k, 1), :] = m
        tid_out_ref[pl.ds(k, 1), :] = minp & 32767
        attn_out_ref[pl.ds(k, 1), :] = (minp >> 15) & 1

    vals = w_ref[...]
    e = jnp.exp(vals - vals[0:1, :])
    w_ref[...] = e / jnp.sum(e, axis=0, keepdims=True)


def kernel(token_id, attn_mask, gate_mask, token_weight):
    b, l = token_weight.shape
    c = min(2048, b)
    grid = (b // c,)

    tw_t = token_weight.T
    tid_t = token_id.T
    gate_t = gate_mask.T
    attn_t = attn_mask.T

    in_spec = pl.BlockSpec((l, c), lambda j: (0, j))
    out_spec = pl.BlockSpec((_K, c), lambda j: (0, j))

    w_t, tid_o, attn_o = pl.pallas_call(
        _topk_body,
        grid=grid,
        in_specs=[in_spec, in_spec, in_spec, in_spec],
        out_specs=[out_spec, out_spec, out_spec],
        out_shape=[
            jax.ShapeDtypeStruct((_K, b), jnp.float32),
            jax.ShapeDtypeStruct((_K, b), jnp.int32),
            jax.ShapeDtypeStruct((_K, b), jnp.int32),
        ],
    )(tw_t, tid_t, gate_t, attn_t)

    return (tid_o.T, attn_o.T, w_t.T)
